# baseline (speedup 1.0000x reference)
import jax
import jax.numpy as jnp
import numpy as np
from jax import lax
from jax.experimental import pallas as pl
from jax.experimental.pallas import tpu as pltpu
from jax.experimental.pallas import tpu_sc as plsc

B = 4096
T = 256
V = 778
ROW = V * 3
NC = 2
NS = 16
NW = NC * NS
RPW = B // NW
C = 4
NCH = RPW // C
L = 16
NG = T // L
NPAIR = 384

_MAGIC = np.int32(0x5F3759DF)


def _sqrt_nr(x):
    i = lax.bitcast_convert_type(x, jnp.int32)
    i = _MAGIC - lax.shift_right_logical(i, 1)
    r = lax.bitcast_convert_type(i, jnp.float32)
    hx = x * np.float32(0.5)
    r = r * (np.float32(1.5) - hx * r * r)
    return x * r


def _edge(x0, x1, x2, y0, y1, y2):
    d0 = x0 - y0
    d1 = x1 - y1
    d2 = x2 - y2
    s = d0 * d0 + d1 * d1 + d2 * d2
    return _sqrt_nr(jnp.maximum(s, np.float32(1e-8)))


def _tri_loss(pbuf, gbuf, rs, ca, cb, cc):
    def lens(buf):
        a = [plsc.load_gather(buf, [rs, ca[k]]) for k in range(3)]
        b = [plsc.load_gather(buf, [rs, cb[k]]) for k in range(3)]
        c = [plsc.load_gather(buf, [rs, cc[k]]) for k in range(3)]
        e1 = _edge(a[0], a[1], a[2], b[0], b[1], b[2])
        e2 = _edge(a[0], a[1], a[2], c[0], c[1], c[2])
        e3 = _edge(b[0], b[1], b[2], c[0], c[1], c[2])
        return e1, e2, e3

    p1, p2, p3 = lens(pbuf)
    g1, g2, g3 = lens(gbuf)
    return jnp.abs(p1 - g1) + jnp.abs(p2 - g2) + jnp.abs(p3 - g3)


def _sc_loss_sums(pred_v, gt_v, fidx):
    mesh = plsc.VectorSubcoreMesh(core_axis_name="c", subcore_axis_name="s")

    @pl.kernel(
        out_type=jax.ShapeDtypeStruct((NW, L), jnp.float32),
        mesh=mesh,
        compiler_params=pltpu.CompilerParams(
            use_tc_tiling_on_sc=False, needs_layout_passes=False),
        scratch_types=[
            pltpu.VMEM((C, V, 3), jnp.float32),
            pltpu.VMEM((C, V, 3), jnp.float32),
            pltpu.VMEM((C, V, 3), jnp.float32),
            pltpu.VMEM((C, V, 3), jnp.float32),
            pltpu.VMEM((C, ROW), jnp.float32),
            pltpu.VMEM((C, ROW), jnp.float32),
            pltpu.VMEM((9, T), jnp.int32),
            pltpu.VMEM((L,), jnp.float32),
            pltpu.SemaphoreType.DMA,
            pltpu.SemaphoreType.DMA,
        ],
    )
    def k(pred_hbm, gt_hbm, fidx_hbm, out_hbm,
          sp0, sg0, sp1, sg1, pc, gc, fidx_v, acc_v, sem0, sem1):
        wid = lax.axis_index("s") * NC + lax.axis_index("c")
        pltpu.sync_copy(fidx_hbm, fidx_v)

        iota = jnp.arange(L, dtype=jnp.int32)
        zero16 = jnp.zeros((L,), jnp.int32)
        kk = iota & 7
        pat01 = lax.shift_right_logical(iota, 3)
        cjv = [pat01 + 2 * j for j in range(8)]
        cmask = kk < 3
        rsplat = [zero16 + r for r in range(C)]

        row0 = wid * RPW
        bufsets = ((sp0, sg0, sem0), (sp1, sg1, sem1))

        def issue(ch, sp, sg, sem):
            base = row0 + ch * C
            pltpu.async_copy(pred_hbm.at[pl.ds(base, C)], sp, sem)
            pltpu.async_copy(gt_hbm.at[pl.ds(base, C)], sg, sem)

        def drain(ch, sp, sg, sem):
            base = row0 + ch * C
            pltpu.make_async_copy(pred_hbm.at[pl.ds(base, C)], sp, sem).wait()
            pltpu.make_async_copy(gt_hbm.at[pl.ds(base, C)], sg, sem).wait()

        issue(0, *bufsets[0])
        issue(1, *bufsets[1])

        def repack(sp, sg):
            def m_body(m8, carry):
                b16 = zero16 + m8 * 16
                for j in range(8):
                    vv = b16 + cjv[j]
                    for r in range(C):
                        o = m8 * 48 + 6 * j
                        xp = plsc.load_gather(sp, [rsplat[r], vv, kk])
                        plsc.store_compressed(
                            pc.at[r, pl.ds(o, L)], xp, mask=cmask)
                        xg = plsc.load_gather(sg, [rsplat[r], vv, kk])
                        plsc.store_compressed(
                            gc.at[r, pl.ds(o, L)], xg, mask=cmask)
                return carry

            lax.fori_loop(0, NPAIR // 8, m_body, 0)

        def compute(acc):
            def g_body(g, acc):
                o = g * L
                ca = [fidx_v[k, pl.ds(o, L)] for k in range(3)]
                cb = [fidx_v[3 + k, pl.ds(o, L)] for k in range(3)]
                cc = [fidx_v[6 + k, pl.ds(o, L)] for k in range(3)]
                parts = [_tri_loss(pc, gc, rsplat[r], ca, cb, cc)
                         for r in range(C)]
                while len(parts) > 1:
                    parts = [a + b for a, b in zip(parts[::2], parts[1::2])]
                return acc + parts[0]

            return lax.fori_loop(0, NG, g_body, acc)

        def pair_body(i, acc):
            for bsel in range(2):
                ch = 2 * i + bsel
                sp, sg, sem = bufsets[bsel]
                drain(ch, sp, sg, sem)
                repack(sp, sg)

                @pl.when(i < NCH // 2 - 1)
                def _():
                    issue(ch + 2, sp, sg, sem)

                acc = compute(acc)
            return acc

        acc = lax.fori_loop(0, NCH // 2, pair_body,
                            jnp.zeros((L,), jnp.float32))
        acc_v[...] = acc
        pltpu.sync_copy(acc_v, out_hbm.at[wid])

    return k(pred_v, gt_v, fidx)


def kernel(pred_v, gt_v, face):
    cols = face.astype(jnp.int32) * 3
    cols9 = cols[:, :, None] + jnp.arange(3, dtype=jnp.int32)[None, None, :]
    fidx = cols9.transpose(1, 2, 0).reshape(9, T)
    sums = _sc_loss_sums(pred_v, gt_v, fidx)
    return jnp.sum(sums) / jnp.float32(3 * T * B)


# speedup vs baseline: 33.2457x; 33.2457x over previous
import jax
import jax.numpy as jnp
import numpy as np
from jax import lax
from jax.experimental import pallas as pl
from jax.experimental.pallas import tpu as pltpu
from jax.experimental.pallas import tpu_sc as plsc

B = 4096
T = 256
ROW = 778 * 3
NC = 2
NS = 16
NW = NC * NS
RPW = B // NW
C = 8
NCH = RPW // C
L = 16
NG = T // L

_MAGIC = np.int32(0x5F3759DF)


def _sqrt_nr(x):
    i = lax.bitcast_convert_type(x, jnp.int32)
    i = _MAGIC - lax.shift_right_logical(i, 1)
    r = lax.bitcast_convert_type(i, jnp.float32)
    hx = x * np.float32(0.5)
    for _ in range(2):
        r = r * (np.float32(1.5) - hx * r * r)
    return x * r


def _edge(x0, x1, x2, y0, y1, y2):
    d0 = x0 - y0
    d1 = x1 - y1
    d2 = x2 - y2
    s = d0 * d0 + d1 * d1 + d2 * d2
    return _sqrt_nr(jnp.maximum(s, np.float32(1e-8)))


def _tri_loss(pbuf, gbuf, rs, ca, cb, cc):
    def lens(buf):
        a = [plsc.load_gather(buf, [rs, ca[k]]) for k in range(3)]
        b = [plsc.load_gather(buf, [rs, cb[k]]) for k in range(3)]
        c = [plsc.load_gather(buf, [rs, cc[k]]) for k in range(3)]
        e1 = _edge(a[0], a[1], a[2], b[0], b[1], b[2])
        e2 = _edge(a[0], a[1], a[2], c[0], c[1], c[2])
        e3 = _edge(b[0], b[1], b[2], c[0], c[1], c[2])
        return e1, e2, e3

    p1, p2, p3 = lens(pbuf)
    g1, g2, g3 = lens(gbuf)
    return jnp.abs(p1 - g1) + jnp.abs(p2 - g2) + jnp.abs(p3 - g3)


def _sc_loss_sums(pred2d, gt2d, fidx):
    mesh = plsc.VectorSubcoreMesh(core_axis_name="c", subcore_axis_name="s")

    @pl.kernel(
        out_type=jax.ShapeDtypeStruct((NW, L), jnp.float32),
        mesh=mesh,
        compiler_params=pltpu.CompilerParams(
            use_tc_tiling_on_sc=False, needs_layout_passes=False),
        scratch_types=[
            pltpu.VMEM((C, ROW), jnp.float32),
            pltpu.VMEM((C, ROW), jnp.float32),
            pltpu.VMEM((9, T), jnp.int32),
            pltpu.VMEM((L,), jnp.float32),
        ],
    )
    def k(pred_hbm, gt_hbm, fidx_hbm, out_hbm, pbuf, gbuf, fidx_v, acc_v):
        wid = lax.axis_index("s") * NC + lax.axis_index("c")
        pltpu.sync_copy(fidx_hbm, fidx_v)

        zero16 = jnp.zeros((L,), jnp.int32)

        def chunk_body(ch, acc):
            base = wid * RPW + ch * C
            pltpu.sync_copy(pred_hbm.at[pl.ds(base, C)], pbuf)
            pltpu.sync_copy(gt_hbm.at[pl.ds(base, C)], gbuf)

            def g_body(g, acc):
                o = g * L
                ca = [fidx_v[k, pl.ds(o, L)] for k in range(3)]
                cb = [fidx_v[3 + k, pl.ds(o, L)] for k in range(3)]
                cc = [fidx_v[6 + k, pl.ds(o, L)] for k in range(3)]

                parts = [_tri_loss(pbuf, gbuf, zero16 + r, ca, cb, cc)
                         for r in range(C)]
                while len(parts) > 1:
                    parts = [a + b for a, b in zip(parts[::2], parts[1::2])]
                return acc + parts[0]

            return lax.fori_loop(0, NG, g_body, acc)

        acc = lax.fori_loop(0, NCH, chunk_body, jnp.zeros((L,), jnp.float32))
        acc_v[...] = acc
        pltpu.sync_copy(acc_v, out_hbm.at[wid])

    return k(pred2d, gt2d, fidx)


def kernel(pred_v, gt_v, face):
    pred2d = pred_v.reshape(B, ROW)
    gt2d = gt_v.reshape(B, ROW)
    cols = face.astype(jnp.int32) * 3
    cols9 = cols[:, :, None] + jnp.arange(3, dtype=jnp.int32)[None, None, :]
    fidx = cols9.transpose(1, 2, 0).reshape(9, T)
    sums = _sc_loss_sums(pred2d, gt2d, fidx)
    return jnp.sum(sums) / jnp.float32(3 * T * B)


# speedup vs baseline: 35.9567x; 1.0815x over previous
import jax
import jax.numpy as jnp
import numpy as np
from jax import lax
from jax.experimental import pallas as pl
from jax.experimental.pallas import tpu as pltpu
from jax.experimental.pallas import tpu_sc as plsc

B = 4096
T = 256
ROW = 778 * 3
NC = 2
NS = 16
NW = NC * NS
RPW = B // NW
C = 8
NCH = RPW // C
L = 16
NG = T // L

_MAGIC = np.int32(0x5F3759DF)


def _sqrt_nr(x):
    i = lax.bitcast_convert_type(x, jnp.int32)
    i = _MAGIC - lax.shift_right_logical(i, 1)
    r = lax.bitcast_convert_type(i, jnp.float32)
    hx = x * np.float32(0.5)
    r = r * (np.float32(1.5) - hx * r * r)
    return x * r


def _edge(x0, x1, x2, y0, y1, y2):
    d0 = x0 - y0
    d1 = x1 - y1
    d2 = x2 - y2
    s = d0 * d0 + d1 * d1 + d2 * d2
    return _sqrt_nr(jnp.maximum(s, np.float32(1e-8)))


def _tri_loss(pbuf, gbuf, rs, ca, cb, cc):
    def lens(buf):
        a = [plsc.load_gather(buf, [rs, ca[k]]) for k in range(3)]
        b = [plsc.load_gather(buf, [rs, cb[k]]) for k in range(3)]
        c = [plsc.load_gather(buf, [rs, cc[k]]) for k in range(3)]
        e1 = _edge(a[0], a[1], a[2], b[0], b[1], b[2])
        e2 = _edge(a[0], a[1], a[2], c[0], c[1], c[2])
        e3 = _edge(b[0], b[1], b[2], c[0], c[1], c[2])
        return e1, e2, e3

    p1, p2, p3 = lens(pbuf)
    g1, g2, g3 = lens(gbuf)
    return jnp.abs(p1 - g1) + jnp.abs(p2 - g2) + jnp.abs(p3 - g3)


def _sc_loss_sums(pred2d, gt2d, fidx):
    mesh = plsc.VectorSubcoreMesh(core_axis_name="c", subcore_axis_name="s")

    @pl.kernel(
        out_type=jax.ShapeDtypeStruct((NW, L), jnp.float32),
        mesh=mesh,
        compiler_params=pltpu.CompilerParams(
            use_tc_tiling_on_sc=False, needs_layout_passes=False),
        scratch_types=[
            pltpu.VMEM((C, ROW), jnp.float32),
            pltpu.VMEM((C, ROW), jnp.float32),
            pltpu.VMEM((C, ROW), jnp.float32),
            pltpu.VMEM((C, ROW), jnp.float32),
            pltpu.VMEM((9, T), jnp.int32),
            pltpu.VMEM((L,), jnp.float32),
            pltpu.SemaphoreType.DMA,
            pltpu.SemaphoreType.DMA,
        ],
    )
    def k(pred_hbm, gt_hbm, fidx_hbm, out_hbm,
          pb0, gb0, pb1, gb1, fidx_v, acc_v, sem0, sem1):
        wid = lax.axis_index("s") * NC + lax.axis_index("c")
        pltpu.sync_copy(fidx_hbm, fidx_v)

        zero16 = jnp.zeros((L,), jnp.int32)
        row0 = wid * RPW
        bufsets = ((pb0, gb0, sem0), (pb1, gb1, sem1))

        def issue(ch, pb, gb, sem):
            base = row0 + ch * C
            pltpu.async_copy(pred_hbm.at[pl.ds(base, C)], pb, sem)
            pltpu.async_copy(gt_hbm.at[pl.ds(base, C)], gb, sem)

        def drain(ch, pb, gb, sem):
            base = row0 + ch * C
            pltpu.make_async_copy(pred_hbm.at[pl.ds(base, C)], pb, sem).wait()
            pltpu.make_async_copy(gt_hbm.at[pl.ds(base, C)], gb, sem).wait()

        issue(0, *bufsets[0])
        issue(1, *bufsets[1])

        def compute(pbuf, gbuf, acc):
            def g_body(g, acc):
                o = g * L
                ca = [fidx_v[k, pl.ds(o, L)] for k in range(3)]
                cb = [fidx_v[3 + k, pl.ds(o, L)] for k in range(3)]
                cc = [fidx_v[6 + k, pl.ds(o, L)] for k in range(3)]

                parts = [_tri_loss(pbuf, gbuf, zero16 + r, ca, cb, cc)
                         for r in range(C)]
                while len(parts) > 1:
                    parts = [a + b for a, b in zip(parts[::2], parts[1::2])]
                return acc + parts[0]

            return lax.fori_loop(0, NG, g_body, acc)

        def pair_body(i, acc):
            for bsel in range(2):
                ch = 2 * i + bsel
                pb, gb, sem = bufsets[bsel]
                drain(ch, pb, gb, sem)
                acc = compute(pb, gb, acc)

                @pl.when(i < NCH // 2 - 1)
                def _():
                    issue(ch + 2, pb, gb, sem)
            return acc

        acc = lax.fori_loop(0, NCH // 2, pair_body,
                            jnp.zeros((L,), jnp.float32))
        acc_v[...] = acc
        pltpu.sync_copy(acc_v, out_hbm.at[wid])

    return k(pred2d, gt2d, fidx)


def kernel(pred_v, gt_v, face):
    pred2d = pred_v.reshape(B, ROW)
    gt2d = gt_v.reshape(B, ROW)
    cols = face.astype(jnp.int32) * 3
    cols9 = cols[:, :, None] + jnp.arange(3, dtype=jnp.int32)[None, None, :]
    fidx = cols9.transpose(1, 2, 0).reshape(9, T)
    sums = _sc_loss_sums(pred2d, gt2d, fidx)
    return jnp.sum(sums) / jnp.float32(3 * T * B)
